# baseline (speedup 1.0000x reference)
import functools

import jax
import jax.numpy as jnp
from jax import lax
from jax.experimental import pallas as pl
from jax.experimental.pallas import tpu as pltpu
from jax.experimental.pallas import tpu_sc as plsc

B = 16384
D = 64
NC = 2
NS = 16
NW = NC * NS
N_PER_W = B // NW
CHUNK = 256
N_CHUNKS = N_PER_W // CHUNK
LANES = 16
TBL = 1000000
NBLK = (TBL + 127) // 128
BLK_PER_W = 245
NPAIR = 123
LIST_CAP = 768
SLOTS = 16
SLOT_CAP = 4096
NV = B // LANES
SCRATCH_ROWS = B + LANES


def _take16(x, idx):
    dnums = lax.GatherDimensionNumbers(
        offset_dims=(), collapsed_slice_dims=(0,), start_index_map=(0,))
    return lax.gather(x, idx[:, None], dnums, slice_sizes=(1,),
                      mode=lax.GatherScatterMode.PROMISE_IN_BOUNDS)


def _sweep_body(table_hbm, rel_hbm, scratch_hbm,
                rel_v, slab_a, slab_b, stage_a, stage_b,
                listc_v, listb_v, slotb_v, slotl_v, cnt_v,
                sem_a, sem_b, esem_a, esem_b):
    lane = lax.iota(jnp.int32, LANES)
    wid = lax.axis_index("s") * NC + lax.axis_index("c")
    blk0 = wid * BLK_PER_W
    c_lo = blk0 * 128
    c_hi = (blk0 + BLK_PER_W) * 128

    pltpu.sync_copy(rel_hbm, rel_v)

    def init_body(i, carry):
        slotb_v[pl.ds(i * LANES, LANES)] = jnp.full((LANES,), -1, jnp.int32)
        return carry

    lax.fori_loop(0, SLOT_CAP // LANES, init_body, 0)

    def init_list(i, carry):
        listc_v[pl.ds(i * LANES, LANES)] = jnp.full((LANES,), -1, jnp.int32)
        return carry

    lax.fori_loop(0, LIST_CAP // LANES, init_list, 0)

    def init_cnt(i, carry):
        cnt_v[pl.ds(i * LANES, LANES)] = jnp.zeros((LANES,), jnp.int32)
        return carry

    lax.fori_loop(0, 256 // LANES, init_cnt, 0)

    def scan_body(i, pos_splat):
        c_vec = rel_v[pl.ds(i * LANES, LANES)]
        b_vec = lane + i * LANES
        mask = (c_vec >= c_lo) & (c_vec < c_hi)
        cs = plsc.cumsum(jnp.where(mask, 1, 0))
        posv = pos_splat + cs - 1
        plsc.store_scatter(listc_v, [posv], c_vec, mask=mask)
        plsc.store_scatter(listb_v, [posv], b_vec, mask=mask)
        return pos_splat + _take16(cs, jnp.full((LANES,), LANES - 1))

    lax.fori_loop(0, NV, scan_body, jnp.zeros((LANES,), jnp.int32))

    def bin_body(j, carry):
        c_vec = listc_v[pl.ds(j * LANES, LANES)]
        b_vec = listb_v[pl.ds(j * LANES, LANES)]
        valid = c_vec >= 0
        blk_loc = jnp.where(valid, (c_vec >> 7) - blk0, 0)
        key = jnp.where(valid, blk_loc, -(lane + 1))
        rank = jnp.zeros((LANES,), jnp.int32)
        later = jnp.zeros((LANES,), jnp.int32)
        for jj in range(1, LANES):
            back = _take16(key, jnp.maximum(lane - jj, 0))
            rank = rank + jnp.where((lane >= jj) & (back == key), 1, 0)
            fwd = _take16(key, jnp.minimum(lane + jj, LANES - 1))
            later = later + jnp.where((lane + jj <= LANES - 1)
                                      & (fwd == key), 1, 0)
        base = plsc.load_gather(cnt_v, [blk_loc])
        slot_idx = blk_loc * SLOTS + base + rank
        plsc.store_scatter(slotb_v, [slot_idx], b_vec, mask=valid)
        plsc.store_scatter(slotl_v, [slot_idx], c_vec & 127, mask=valid)
        is_last = valid & (later == 0)
        plsc.store_scatter(cnt_v, [blk_loc], base + rank + 1, mask=is_last)
        return carry

    lax.fori_loop(0, LIST_CAP // LANES, bin_body, 0)

    def blk_of(g):
        return jnp.minimum(blk0 + g, NBLK - 1)

    def fire_slab(g, slab, sem):
        return pltpu.async_copy(
            table_hbm.at[:, pl.ds(blk_of(g) * 128, 128)], slab, sem)

    def prime_emit(stage, esem):
        for m in range(SLOTS):
            pltpu.async_copy(stage.at[pl.ds(m, 1), :],
                             scratch_hbm.at[pl.ds(B + m, 1), :], esem)

    def drain_emit(stage, esem):
        for m in range(SLOTS):
            pltpu.make_async_copy(stage.at[pl.ds(m, 1), :],
                                  scratch_hbm.at[pl.ds(B + m, 1), :],
                                  esem).wait()

    def process(g, slab, stage, esem):
        slot_b = slotb_v[pl.ds(g * SLOTS, SLOTS)]
        slot_l = slotl_v[pl.ds(g * SLOTS, SLOTS)]
        lanes_cl = jnp.where(slot_b >= 0, slot_l, 0)
        for d in range(D):
            dv = jnp.full((LANES,), d, jnp.int32)
            vals = plsc.load_gather(slab, [dv, lanes_cl])
            plsc.store_scatter(stage, [lane, dv], vals)
        for m in range(SLOTS):
            b_s = jnp.sum(jnp.where(lane == m, slot_b, 0))
            b_safe = jnp.where(b_s >= 0, b_s, B + m)
            pltpu.async_copy(stage.at[pl.ds(m, 1), :],
                             scratch_hbm.at[pl.ds(b_safe, 1), :], esem)

    fire_slab(0, slab_a, sem_a)
    prime_emit(stage_a, esem_a)
    prime_emit(stage_b, esem_b)

    def pair_body(k, carry):
        g0 = 2 * k
        fire_slab(g0 + 1, slab_b, sem_b)
        pltpu.make_async_copy(
            table_hbm.at[:, pl.ds(blk_of(g0) * 128, 128)],
            slab_a, sem_a).wait()
        drain_emit(stage_a, esem_a)
        process(g0, slab_a, stage_a, esem_a)
        fire_slab(g0 + 2, slab_a, sem_a)
        pltpu.make_async_copy(
            table_hbm.at[:, pl.ds(blk_of(g0 + 1) * 128, 128)],
            slab_b, sem_b).wait()
        drain_emit(stage_b, esem_b)
        process(g0 + 1, slab_b, stage_b, esem_b)
        return carry

    lax.fori_loop(0, NPAIR, pair_body, 0)
    pltpu.make_async_copy(
        table_hbm.at[:, pl.ds(blk_of(2 * NPAIR) * 128, 128)],
        slab_a, sem_a).wait()
    drain_emit(stage_a, esem_a)
    drain_emit(stage_b, esem_b)


@functools.partial(
    pl.kernel,
    out_type=jax.ShapeDtypeStruct((SCRATCH_ROWS, D), jnp.float32),
    mesh=plsc.VectorSubcoreMesh(core_axis_name="c", subcore_axis_name="s"),
    compiler_params=pltpu.CompilerParams(needs_layout_passes=False),
    scratch_types=[
        pltpu.VMEM((B,), jnp.int32),
        pltpu.VMEM((D, 128), jnp.float32),
        pltpu.VMEM((D, 128), jnp.float32),
        pltpu.VMEM((SLOTS, D), jnp.float32),
        pltpu.VMEM((SLOTS, D), jnp.float32),
        pltpu.VMEM((LIST_CAP,), jnp.int32),
        pltpu.VMEM((LIST_CAP,), jnp.int32),
        pltpu.VMEM((SLOT_CAP,), jnp.int32),
        pltpu.VMEM((SLOT_CAP,), jnp.int32),
        pltpu.VMEM((256,), jnp.int32),
        pltpu.SemaphoreType.DMA,
        pltpu.SemaphoreType.DMA,
        pltpu.SemaphoreType.DMA,
        pltpu.SemaphoreType.DMA,
    ],
)
def _gather_sweep(table_hbm, rel_hbm, scratch_hbm,
                  rel_v, slab_a, slab_b, stage_a, stage_b,
                  listc_v, listb_v, slotb_v, slotl_v, cnt_v,
                  sem_a, sem_b, esem_a, esem_b):
    _sweep_body(table_hbm, rel_hbm, scratch_hbm,
                rel_v, slab_a, slab_b, stage_a, stage_b,
                listc_v, listb_v, slotb_v, slotl_v, cnt_v,
                sem_a, sem_b, esem_a, esem_b)


def _compute_body(rows_hbm, h_hbm, r_hbm, t_hbm, out_hbm,
                  rows_v, h_v, r_v, t_v, out_v, sem, gsem):
    wid = lax.axis_index("s") * NC + lax.axis_index("c")
    base = wid * N_PER_W
    for c in range(N_CHUNKS):
        col0 = base + c * CHUNK
        grows = pltpu.async_copy(
            rows_hbm.at[pl.ds(col0, CHUNK), :], rows_v, gsem)
        dense = [
            pltpu.async_copy(h_hbm.at[:, pl.ds(col0, CHUNK)], h_v, sem),
            pltpu.async_copy(r_hbm.at[:, pl.ds(col0, CHUNK)], r_v, sem),
            pltpu.async_copy(t_hbm.at[:, pl.ds(col0, CHUNK)], t_v, sem),
        ]
        for cp in dense:
            cp.wait()
        grows.wait()

        def group_body(g, carry, c=c):
            sl = pl.ds(g * LANES, LANES)
            bvec = lax.iota(jnp.int32, LANES) + g * LANES
            acc = jnp.zeros((LANES,), jnp.float32)
            for d in range(D):
                dvec = jnp.full((LANES,), d, jnp.int32)
                gv = plsc.load_gather(rows_v, [bvec, dvec])
                acc = acc + h_v[d, sl] * t_v[d, sl] * (r_v[d, sl] + gv)
            out_v[pl.ds(c * CHUNK + g * LANES, LANES)] = acc * 0.5
            return carry

        lax.fori_loop(0, CHUNK // LANES, group_body, 0)
    pltpu.sync_copy(out_v, out_hbm.at[pl.ds(base, N_PER_W)])


@functools.partial(
    pl.kernel,
    out_type=jax.ShapeDtypeStruct((B,), jnp.float32),
    mesh=plsc.VectorSubcoreMesh(core_axis_name="c", subcore_axis_name="s"),
    compiler_params=pltpu.CompilerParams(needs_layout_passes=False),
    scratch_types=[
        pltpu.VMEM((CHUNK, D), jnp.float32),
        pltpu.VMEM((D, CHUNK), jnp.float32),
        pltpu.VMEM((D, CHUNK), jnp.float32),
        pltpu.VMEM((D, CHUNK), jnp.float32),
        pltpu.VMEM((N_PER_W,), jnp.float32),
        pltpu.SemaphoreType.DMA,
        pltpu.SemaphoreType.DMA,
    ],
)
def _simple_cal_compute(rows_hbm, h_hbm, r_hbm, t_hbm, out_hbm,
                        rows_v, h_v, r_v, t_v, out_v, sem, gsem):
    _compute_body(rows_hbm, h_hbm, r_hbm, t_hbm, out_hbm,
                  rows_v, h_v, r_v, t_v, out_v, sem, gsem)


def kernel(x0, x1, x2, rel, rel_inv_table):
    h = x0.reshape(B, D).T
    r = x1.reshape(B, D).T
    t = x2.reshape(B, D).T
    table = rel_inv_table.T
    scratch = _gather_sweep(table, rel)
    out = _simple_cal_compute(scratch, h, r, t)
    return out[:, None]
